# baseline (speedup 1.0000x reference)
import jax
import jax.numpy as jnp
from jax.experimental import pallas as pl


def _copy_body(x_ref, o_ref):
    o_ref[...] = x_ref[...]


def _pallas_copy(joints):
    B, J, C = joints.shape
    flat = joints.reshape(B, J * C)
    out = pl.pallas_call(
        _copy_body,
        out_shape=jax.ShapeDtypeStruct((B, J * C), flat.dtype),
    )(flat)
    return out.reshape(B, J, C)


def _gather_concat_body(idx_ref, verts_ref, joints_ref, o_ref):
    J = joints_ref.shape[1]
    K = idx_ref.shape[0]
    o_ref[0, :J, :] = joints_ref[0, :, :]
    for k in range(K):
        o_ref[0, J + k, :] = verts_ref[0, idx_ref[k], :]


def kernel(vertices, joints, extra_joints_idxs):
    K = extra_joints_idxs.shape[0]
    if K == 0:
        return _pallas_copy(joints)

    B, J, C = joints.shape
    V = vertices.shape[1]
    from jax.experimental.pallas import tpu as pltpu

    return pl.pallas_call(
        _gather_concat_body,
        grid_spec=pltpu.PrefetchScalarGridSpec(
            num_scalar_prefetch=1,
            grid=(B,),
            in_specs=[
                pl.BlockSpec((1, V, C), lambda b, idx: (b, 0, 0)),
                pl.BlockSpec((1, J, C), lambda b, idx: (b, 0, 0)),
            ],
            out_specs=pl.BlockSpec((1, J + K, C), lambda b, idx: (b, 0, 0)),
        ),
        out_shape=jax.ShapeDtypeStruct((B, J + K, C), joints.dtype),
    )(extra_joints_idxs, vertices, joints)
